# baseline (speedup 1.0000x reference)
import functools
import math

import jax
import jax.numpy as jnp
from jax import lax
from jax.experimental import pallas as pl
from jax.experimental.pallas import tpu as pltpu
from jax.experimental.pallas import tpu_sc as plsc

N_SAMPLES = 4_194_304
N_HIT = 8192
N_RAYS = 16384
SIGMA = 1.0
SIGMA_SCALE = 3.0
STD = SIGMA / SIGMA_SCALE
INV2STD2 = 1.0 / (2.0 * STD * STD)
PDF_C = 1.0 / (STD * math.sqrt(2.0 * math.pi))

NC = 2
NS = 16
L = 16
NW = NC * NS
CHUNK = N_SAMPLES // NW
BLK = 4096
RING = 4
NBLK = CHUNK // BLK
HIT_PER_W = N_HIT // NW
UNROLL = 8


def _log(x):
    xi = plsc.bitcast(x, jnp.int32)
    y = xi.astype(jnp.float32) * 8.262958405176314e-8 - 87.98997108999257
    for _ in range(3):
        y = y - 1.0 + x * jnp.exp(-y)
    return y


_mesh = plsc.VectorSubcoreMesh(
    core_axis_name="c", subcore_axis_name="s", num_cores=NC, num_subcores=NS
)

_scratch = (
    [
        pltpu.VMEM((N_RAYS,), jnp.float32),
        pltpu.VMEM((N_RAYS,), jnp.float32),
        pltpu.VMEM((N_HIT,), jnp.int32),
        pltpu.VMEM((HIT_PER_W,), jnp.float32),
        pltpu.VMEM((N_HIT,), jnp.float32),
        pltpu.VMEM((4, L), jnp.float32),
    ]
    + [pltpu.VMEM((BLK,), jnp.float32) for _ in range(RING)]
    + [pltpu.VMEM((BLK,), jnp.float32) for _ in range(RING)]
    + [pltpu.VMEM((BLK,), jnp.int32) for _ in range(RING)]
    + [pltpu.SemaphoreType.DMA for _ in range(RING)]
    + [pltpu.SemaphoreType.DMA]
)


@functools.partial(
    pl.kernel,
    out_type=jax.ShapeDtypeStruct((NW, 4, L), jnp.float32),
    mesh=_mesh,
    compiler_params=pltpu.CompilerParams(needs_layout_passes=False),
    scratch_types=_scratch,
)
def _lidar_sc(t_hbm, vw_hbm, ranges_hbm, dv_hbm, seg_hbm, rhit_hbm, maskf_hbm,
              out_hbm,
              ranges_v, maskf_v, rhit_v, dv_v, gp_v, outs_v, *scr):
    t_bufs = scr[0:RING]
    vw_bufs = scr[RING:2 * RING]
    seg_bufs = scr[2 * RING:3 * RING]
    sems = scr[3 * RING:4 * RING]
    semt = scr[4 * RING]

    wid = lax.axis_index("s") * NC + lax.axis_index("c")
    samp_base = wid * CHUNK
    hit_base = wid * HIT_PER_W

    def start_blk(j, slot):
        off = samp_base + j * BLK
        pltpu.async_copy(t_hbm.at[pl.ds(off, BLK)], t_bufs[slot], sems[slot])
        pltpu.async_copy(vw_hbm.at[pl.ds(off, BLK)], vw_bufs[slot], sems[slot])
        pltpu.async_copy(seg_hbm.at[pl.ds(off, BLK)], seg_bufs[slot], sems[slot])

    def wait_blk(slot):
        pltpu.make_async_copy(t_hbm.at[pl.ds(0, BLK)], t_bufs[slot], sems[slot]).wait()
        pltpu.make_async_copy(vw_hbm.at[pl.ds(0, BLK)], vw_bufs[slot], sems[slot]).wait()
        pltpu.make_async_copy(seg_hbm.at[pl.ds(0, BLK)], seg_bufs[slot], sems[slot]).wait()

    c1 = pltpu.async_copy(ranges_hbm, ranges_v, semt)
    c2 = pltpu.async_copy(maskf_hbm, maskf_v, semt)
    c3 = pltpu.async_copy(rhit_hbm, rhit_v, semt)
    c4 = pltpu.async_copy(dv_hbm.at[pl.ds(hit_base, HIT_PER_W)], dv_v, semt)
    for s in range(RING):
        start_blk(s, s)
    c1.wait(); c2.wait(); c3.wait(); c4.wait()

    def tbl_body(i):
        sl = pl.ds(i, L)
        ridx = rhit_v[sl]
        g = plsc.load_gather(ranges_v, [ridx])
        m = plsc.load_gather(maskf_v, [ridx])
        gp_v[sl] = jnp.where(m > 0.5, g, -1e9)

    plsc.parallel_loop(0, N_HIT, step=L)(tbl_body)

    def depth_body(i, accs):
        accd, accm = accs
        ridx = rhit_v[pl.ds(hit_base + i, L)]
        g = plsc.load_gather(ranges_v, [ridx])
        m = plsc.load_gather(maskf_v, [ridx])
        dvv = dv_v[pl.ds(i, L)]
        g_safe = jnp.where(m > 0.5, g, 1.0)
        d = jnp.abs(_log(dvv + 1.0) - _log(g_safe + 1.0)) * m
        return accd + d, accm + m

    zero = jnp.zeros((L,), jnp.float32)
    accd, accm = plsc.parallel_loop(0, HIT_PER_W, step=L, carry=(zero, zero))(depth_body)

    def compute_blk(slot, accs):
        tb = t_bufs[slot]
        vb = vw_bufs[slot]
        sb = seg_bufs[slot]

        def vec_body(i, accs2):
            accs3 = list(accs2)
            for u in range(UNROLL):
                sl = pl.ds(i + u * L, L)
                seg = sb[sl]
                gp = plsc.load_gather(gp_v, [seg])
                tt = tb[sl]
                vv = vb[sl]
                diff = tt - gp
                d2 = diff * diff
                p = PDF_C * jnp.exp(d2 * (-INV2STD2))
                r = vv - p
                nb = jnp.where(d2 <= SIGMA * SIGMA, r * r, 0.0)
                eb = jnp.where(diff < -SIGMA, vv * vv, 0.0)
                accs3[2 * u] = accs3[2 * u] + nb
                accs3[2 * u + 1] = accs3[2 * u + 1] + eb
            return tuple(accs3)

        return plsc.parallel_loop(0, BLK, step=L * UNROLL, carry=tuple(accs))(vec_body)

    def blk_body(k, accs):
        for slot in range(RING):
            j = k * RING + slot
            wait_blk(slot)
            accs = compute_blk(slot, accs)

            @pl.when(j + RING < NBLK)
            def _():
                start_blk(j + RING, slot)

        return accs

    accs = tuple([zero] * (2 * UNROLL))
    accs = lax.fori_loop(0, NBLK // RING, blk_body, accs)
    accn = accs[0]
    acce = accs[1]
    for u in range(1, UNROLL):
        accn = accn + accs[2 * u]
        acce = acce + accs[2 * u + 1]

    outs_v[0, :] = accn
    outs_v[1, :] = acce
    outs_v[2, :] = accd
    outs_v[3, :] = accm
    pltpu.sync_copy(outs_v, out_hbm.at[wid])


def kernel(t, vw, ranges, depth_volume, segment_ids, rays_inds_hit, mask):
    seg = segment_ids.astype(jnp.int32)
    rhit = rays_inds_hit.astype(jnp.int32)
    maskf = mask.astype(jnp.float32)
    parts = _lidar_sc(t, vw, ranges, depth_volume, seg, rhit, maskf)
    s = jnp.sum(parts, axis=(0, 2))
    depth_loss = s[2] / jnp.maximum(s[3], 1.0)
    neighbor_loss = s[0] / N_HIT
    empty_loss = s[1] / N_HIT
    return jnp.stack([depth_loss, neighbor_loss, empty_loss])


# speedup vs baseline: 1.0437x; 1.0437x over previous
import functools
import math

import jax
import jax.numpy as jnp
from jax import lax
from jax.experimental import pallas as pl
from jax.experimental.pallas import tpu as pltpu
from jax.experimental.pallas import tpu_sc as plsc

N_SAMPLES = 4_194_304
N_HIT = 8192
N_RAYS = 16384
SIGMA = 1.0
SIGMA_SCALE = 3.0
STD = SIGMA / SIGMA_SCALE
INV2STD2 = 1.0 / (2.0 * STD * STD)
PDF_C = 1.0 / (STD * math.sqrt(2.0 * math.pi))

NC = 2
NS = 16
L = 16
NW = NC * NS
CHUNK = N_SAMPLES // NW
BLK = 8192
RING = 4
NBLK = CHUNK // BLK
HIT_PER_T = N_HIT // NS
IDX_CHUNK = 128
NIC = HIT_PER_T // IDX_CHUNK
UNROLL = 8


def _log(x):
    xi = plsc.bitcast(x, jnp.int32)
    y = xi.astype(jnp.float32) * 8.262958405176314e-8 - 87.98997108999257
    for _ in range(3):
        y = y - 1.0 + x * jnp.exp(-y)
    return y


_mesh = plsc.VectorSubcoreMesh(
    core_axis_name="c", subcore_axis_name="s", num_cores=NC, num_subcores=NS
)

_scratch = (
    [
        pltpu.VMEM((N_HIT,), jnp.float32),
        pltpu.VMEM((HIT_PER_T,), jnp.float32),
        pltpu.VMEM((4, L), jnp.float32),
        pltpu.VMEM_SHARED((N_HIT,), jnp.float32),
    ]
    + [pltpu.VMEM((IDX_CHUNK,), jnp.int32) for _ in range(NIC)]
    + [pltpu.VMEM((IDX_CHUNK,), jnp.float32) for _ in range(NIC)]
    + [pltpu.VMEM((IDX_CHUNK,), jnp.float32) for _ in range(NIC)]
    + [pltpu.VMEM((BLK,), jnp.float32) for _ in range(RING)]
    + [pltpu.VMEM((BLK,), jnp.float32) for _ in range(RING)]
    + [pltpu.VMEM((BLK,), jnp.int32) for _ in range(RING)]
    + [pltpu.SemaphoreType.DMA for _ in range(RING)]
    + [pltpu.SemaphoreType.DMA]
)


@functools.partial(
    pl.kernel,
    out_type=jax.ShapeDtypeStruct((NW, 4, L), jnp.float32),
    mesh=_mesh,
    compiler_params=pltpu.CompilerParams(needs_layout_passes=False),
    scratch_types=_scratch,
)
def _lidar_sc(t_hbm, vw_hbm, ranges_hbm, dv_hbm, seg_hbm, rhit_hbm, maskf_hbm,
              out_hbm,
              gp_v, dv_v, outs_v, gp_sh, *scr):
    rhit_c = scr[0:NIC]
    g_c = scr[NIC:2 * NIC]
    m_c = scr[2 * NIC:3 * NIC]
    base = 3 * NIC
    t_bufs = scr[base:base + RING]
    vw_bufs = scr[base + RING:base + 2 * RING]
    seg_bufs = scr[base + 2 * RING:base + 3 * RING]
    sems = scr[base + 3 * RING:base + 4 * RING]
    semt = scr[base + 4 * RING]

    tid = lax.axis_index("s")
    core = lax.axis_index("c")
    wid = tid * NC + core
    samp_base = wid * CHUNK
    hit_base = tid * HIT_PER_T

    def start_blk(j, slot):
        off = samp_base + j * BLK
        pltpu.async_copy(t_hbm.at[pl.ds(off, BLK)], t_bufs[slot], sems[slot])
        pltpu.async_copy(vw_hbm.at[pl.ds(off, BLK)], vw_bufs[slot], sems[slot])
        pltpu.async_copy(seg_hbm.at[pl.ds(off, BLK)], seg_bufs[slot], sems[slot])

    def wait_blk(slot):
        pltpu.make_async_copy(t_hbm.at[pl.ds(0, BLK)], t_bufs[slot], sems[slot]).wait()
        pltpu.make_async_copy(vw_hbm.at[pl.ds(0, BLK)], vw_bufs[slot], sems[slot]).wait()
        pltpu.make_async_copy(seg_hbm.at[pl.ds(0, BLK)], seg_bufs[slot], sems[slot]).wait()

    hc = [
        pltpu.async_copy(
            rhit_hbm.at[pl.ds(hit_base + k * IDX_CHUNK, IDX_CHUNK)], rhit_c[k], semt
        )
        for k in range(NIC)
    ]
    cdv = pltpu.async_copy(dv_hbm.at[pl.ds(hit_base, HIT_PER_T)], dv_v, semt)
    for s in range(RING):
        start_blk(s, s)
    for c in hc:
        c.wait()

    gc = [
        pltpu.async_copy(ranges_hbm.at[rhit_c[k]], g_c[k], semt) for k in range(NIC)
    ] + [
        pltpu.async_copy(maskf_hbm.at[rhit_c[k]], m_c[k], semt) for k in range(NIC)
    ]
    for c in gc:
        c.wait()
    cdv.wait()

    for k in range(NIC):
        gk, mk = g_c[k], m_c[k]

        def mask_body(i, gk=gk, mk=mk):
            sl = pl.ds(i, L)
            gk[sl] = jnp.where(mk[sl] > 0.5, gk[sl], -1e9)

        plsc.parallel_loop(0, IDX_CHUNK, step=L)(mask_body)
        pltpu.sync_copy(g_c[k], gp_sh.at[pl.ds(hit_base + k * IDX_CHUNK, IDX_CHUNK)])

    plsc.subcore_barrier()
    pltpu.sync_copy(gp_sh, gp_v)

    zero = jnp.zeros((L,), jnp.float32)
    accd, accm = zero, zero
    for k in range(NIC):
        gk, mk = g_c[k], m_c[k]

        def depth_body(i, accs, gk=gk, mk=mk, k=k):
            acd, acm = accs
            sl = pl.ds(i, L)
            m = mk[sl]
            gp = gk[sl]
            dvv = dv_v[pl.ds(k * IDX_CHUNK + i, L)]
            g_safe = jnp.where(m > 0.5, gp, 1.0)
            d = jnp.abs(_log(dvv + 1.0) - _log(g_safe + 1.0)) * m
            return acd + d, acm + m

        accd, accm = plsc.parallel_loop(
            0, IDX_CHUNK, step=L, carry=(accd, accm)
        )(depth_body)
    accd = accd * 0.5
    accm = accm * 0.5

    def compute_blk(slot, accs):
        tb = t_bufs[slot]
        vb = vw_bufs[slot]
        sb = seg_bufs[slot]

        def vec_body(i, accs2):
            accs3 = list(accs2)
            for u in range(UNROLL):
                sl = pl.ds(i + u * L, L)
                seg = sb[sl]
                gp = plsc.load_gather(gp_v, [seg])
                tt = tb[sl]
                vv = vb[sl]
                diff = tt - gp
                d2 = diff * diff
                p = PDF_C * jnp.exp(d2 * (-INV2STD2))
                r = vv - p
                nb = jnp.where(d2 <= SIGMA * SIGMA, r * r, 0.0)
                eb = jnp.where(diff < -SIGMA, vv * vv, 0.0)
                accs3[2 * u] = accs3[2 * u] + nb
                accs3[2 * u + 1] = accs3[2 * u + 1] + eb
            return tuple(accs3)

        return plsc.parallel_loop(0, BLK, step=L * UNROLL, carry=tuple(accs))(vec_body)

    def blk_body(k, accs):
        for slot in range(RING):
            j = k * RING + slot
            wait_blk(slot)
            accs = compute_blk(slot, accs)

            @pl.when(j + RING < NBLK)
            def _():
                start_blk(j + RING, slot)

        return accs

    accs = tuple([zero] * (2 * UNROLL))
    accs = lax.fori_loop(0, NBLK // RING, blk_body, accs)
    accn = accs[0]
    acce = accs[1]
    for u in range(1, UNROLL):
        accn = accn + accs[2 * u]
        acce = acce + accs[2 * u + 1]

    outs_v[0, :] = accn
    outs_v[1, :] = acce
    outs_v[2, :] = accd
    outs_v[3, :] = accm
    pltpu.sync_copy(outs_v, out_hbm.at[wid])


def kernel(t, vw, ranges, depth_volume, segment_ids, rays_inds_hit, mask):
    seg = segment_ids.astype(jnp.int32)
    rhit = rays_inds_hit.astype(jnp.int32)
    maskf = mask.astype(jnp.float32)
    parts = _lidar_sc(t, vw, ranges, depth_volume, seg, rhit, maskf)
    s = jnp.sum(parts, axis=(0, 2))
    depth_loss = s[2] / jnp.maximum(s[3], 1.0)
    neighbor_loss = s[0] / N_HIT
    empty_loss = s[1] / N_HIT
    return jnp.stack([depth_loss, neighbor_loss, empty_loss])
